# baseline (speedup 1.0000x reference)
import numpy as np
import jax
import jax.numpy as jnp
from jax.experimental import pallas as pl
from jax.experimental.pallas import tpu as pltpu

B = 32
V = 1000000
BLK = 8192
C = 512
NCHUNK = BLK // C
NBLK = (V + BLK - 1) // BLK

_KS1 = 42
_KS2 = 42 ^ 0x1BD11BDA
_ROTS = ((13, 15, 26, 6), (17, 29, 16, 24))
_KS = (0, _KS1, _KS2)
_TINY = np.float32(np.finfo(np.float32).tiny)


def _threefry_bits(flat):
    x1 = flat + jnp.uint32(_KS1)
    x0 = x1
    x1 = (x1 << jnp.uint32(13)) | (x1 >> jnp.uint32(32 - 13))
    x1 = x1 ^ x0
    first = True
    for i in range(5):
        for r in _ROTS[i % 2]:
            if first:
                first = False
                continue
            x0 = x0 + x1
            x1 = (x1 << jnp.uint32(r)) | (x1 >> jnp.uint32(32 - r))
            x1 = x1 ^ x0
        x0 = x0 + jnp.uint32(_KS[(i + 1) % 3])
        x1 = x1 + jnp.uint32((_KS[(i + 2) % 3] + (i + 1)) & 0xFFFFFFFF)
    return x0 ^ x1


def _sample_kernel(logits_ref, temps_ref, out_ref, bv_ref, bi_ref):
    j = pl.program_id(0)

    @pl.when(j == 0)
    def _init():
        bv_ref[...] = jnp.full((B, C), -jnp.inf, jnp.float32)
        bi_ref[...] = jnp.zeros((B, C), jnp.int32)

    t = temps_ref[...]
    iota_c = jax.lax.broadcasted_iota(jnp.int32, (B, C), 1)
    row_v = (jax.lax.broadcasted_iota(jnp.uint32, (B, 1), 0)
             * jnp.uint32(V))
    base0 = j * BLK

    def chunk(k, carry, masked):
        bv, bi = carry
        col = iota_c + (base0 + k * C)
        flat = col.astype(jnp.uint32) + row_v

        bits = _threefry_bits(flat)
        fb = (bits >> jnp.uint32(9)) | jnp.uint32(0x3F800000)
        floats = jax.lax.bitcast_convert_type(fb, jnp.float32) - jnp.float32(1.0)
        u = floats + _TINY
        lg = logits_ref[:, pl.ds(k * C, C)]
        val = lg / t - jnp.log(-jnp.log(u))
        if masked:
            val = jnp.where(col < V, val, jnp.float32(-jnp.inf))

        upd = val > bv
        return jnp.where(upd, val, bv), jnp.where(upd, col, bi)

    carry0 = (bv_ref[...], bi_ref[...])
    bv, bi = jax.lax.cond(
        j == NBLK - 1,
        lambda c: jax.lax.fori_loop(
            0, NCHUNK, lambda k, cc: chunk(k, cc, True), c, unroll=16),
        lambda c: jax.lax.fori_loop(
            0, NCHUNK, lambda k, cc: chunk(k, cc, False), c, unroll=16),
        carry0)
    bv_ref[...] = bv
    bi_ref[...] = bi

    @pl.when(j == NBLK - 1)
    def _fin():
        m = jnp.max(bv, axis=1, keepdims=True)
        cand = jnp.where(bv == m, bi, jnp.int32(0x7FFFFFFF))
        out_ref[...] = jnp.min(cand, axis=1, keepdims=True)


def kernel(logits, temperatures):
    out = pl.pallas_call(
        _sample_kernel,
        grid=(NBLK,),
        in_specs=[
            pl.BlockSpec((B, BLK), lambda j: (0, j)),
            pl.BlockSpec((B, 1), lambda j: (0, 0)),
        ],
        out_specs=pl.BlockSpec((B, 1), lambda j: (0, 0)),
        out_shape=jax.ShapeDtypeStruct((B, 1), jnp.int32),
        scratch_shapes=[
            pltpu.VMEM((B, C), jnp.float32),
            pltpu.VMEM((B, C), jnp.int32),
        ],
        compiler_params=pltpu.CompilerParams(
            dimension_semantics=("arbitrary",),
        ),
    )(logits, temperatures.reshape(B, 1))
    return out.reshape(B)


# speedup vs baseline: 1.0396x; 1.0396x over previous
import numpy as np
import jax
import jax.numpy as jnp
from jax.experimental import pallas as pl
from jax.experimental.pallas import tpu as pltpu

B = 32
V = 1000000
BLK = 8192
C = 256
NCHUNK = BLK // C
NBLK = (V + BLK - 1) // BLK

_KS1 = 42
_KS2 = 42 ^ 0x1BD11BDA
_ROTS = ((13, 15, 26, 6), (17, 29, 16, 24))
_KS = (0, _KS1, _KS2)
_TINY = np.float32(np.finfo(np.float32).tiny)


def _threefry_bits(flat):
    x1 = flat + jnp.uint32(_KS1)
    x0 = x1
    x1 = (x1 << jnp.uint32(13)) | (x1 >> jnp.uint32(32 - 13))
    x1 = x1 ^ x0
    first = True
    for i in range(5):
        for r in _ROTS[i % 2]:
            if first:
                first = False
                continue
            x0 = x0 + x1
            x1 = (x1 << jnp.uint32(r)) | (x1 >> jnp.uint32(32 - r))
            x1 = x1 ^ x0
        x0 = x0 + jnp.uint32(_KS[(i + 1) % 3])
        x1 = x1 + jnp.uint32((_KS[(i + 2) % 3] + (i + 1)) & 0xFFFFFFFF)
    return x0 ^ x1


def _sample_kernel(logits_ref, temps_ref, out_ref, bv_ref, bi_ref):
    j = pl.program_id(0)

    @pl.when(j == 0)
    def _init():
        bv_ref[...] = jnp.full((B, C), -jnp.inf, jnp.float32)
        bi_ref[...] = jnp.zeros((B, C), jnp.int32)

    t = temps_ref[...]
    iota_c = jax.lax.broadcasted_iota(jnp.int32, (B, C), 1)
    row_v = (jax.lax.broadcasted_iota(jnp.uint32, (B, 1), 0)
             * jnp.uint32(V))
    base0 = j * BLK

    def chunk(k, carry, masked):
        bv, bi = carry
        col = iota_c + (base0 + k * C)
        flat = col.astype(jnp.uint32) + row_v

        bits = _threefry_bits(flat)
        fb = (bits >> jnp.uint32(9)) | jnp.uint32(0x3F800000)
        floats = jax.lax.bitcast_convert_type(fb, jnp.float32) - jnp.float32(1.0)
        u = floats + _TINY
        lg = logits_ref[:, pl.ds(k * C, C)]
        val = lg / t - jnp.log(-jnp.log(u))
        if masked:
            val = jnp.where(col < V, val, jnp.float32(-jnp.inf))

        upd = val > bv
        return jnp.where(upd, val, bv), jnp.where(upd, col, bi)

    carry0 = (bv_ref[...], bi_ref[...])
    bv, bi = jax.lax.cond(
        j == NBLK - 1,
        lambda c: jax.lax.fori_loop(
            0, NCHUNK, lambda k, cc: chunk(k, cc, True), c, unroll=32),
        lambda c: jax.lax.fori_loop(
            0, NCHUNK, lambda k, cc: chunk(k, cc, False), c, unroll=32),
        carry0)
    bv_ref[...] = bv
    bi_ref[...] = bi

    @pl.when(j == NBLK - 1)
    def _fin():
        m = jnp.max(bv, axis=1, keepdims=True)
        cand = jnp.where(bv == m, bi, jnp.int32(0x7FFFFFFF))
        out_ref[...] = jnp.min(cand, axis=1, keepdims=True)


def kernel(logits, temperatures):
    out = pl.pallas_call(
        _sample_kernel,
        grid=(NBLK,),
        in_specs=[
            pl.BlockSpec((B, BLK), lambda j: (0, j)),
            pl.BlockSpec((B, 1), lambda j: (0, 0)),
        ],
        out_specs=pl.BlockSpec((B, 1), lambda j: (0, 0)),
        out_shape=jax.ShapeDtypeStruct((B, 1), jnp.int32),
        scratch_shapes=[
            pltpu.VMEM((B, C), jnp.float32),
            pltpu.VMEM((B, C), jnp.int32),
        ],
        compiler_params=pltpu.CompilerParams(
            dimension_semantics=("arbitrary",),
        ),
    )(logits, temperatures.reshape(B, 1))
    return out.reshape(B)


# speedup vs baseline: 1.0430x; 1.0032x over previous
import numpy as np
import jax
import jax.numpy as jnp
from jax.experimental import pallas as pl
from jax.experimental.pallas import tpu as pltpu

B = 32
V = 1000000
BLK = 8192
C = 256
NCHUNK = BLK // C
NBLK = (V + BLK - 1) // BLK

_KS1 = 42
_KS2 = 42 ^ 0x1BD11BDA
_ROTS = ((13, 15, 26, 6), (17, 29, 16, 24))
_KS = (0, _KS1, _KS2)
_TINY = np.float32(np.finfo(np.float32).tiny)


def _threefry_bits(flat):
    x1 = flat + jnp.uint32(_KS1)
    x0 = x1
    x1 = (x1 << jnp.uint32(13)) | (x1 >> jnp.uint32(32 - 13))
    x1 = x1 ^ x0
    first = True
    for i in range(5):
        for r in _ROTS[i % 2]:
            if first:
                first = False
                continue
            x0 = x0 + x1
            x1 = (x1 << jnp.uint32(r)) | (x1 >> jnp.uint32(32 - r))
            x1 = x1 ^ x0
        x0 = x0 + jnp.uint32(_KS[(i + 1) % 3])
        x1 = x1 + jnp.uint32((_KS[(i + 2) % 3] + (i + 1)) & 0xFFFFFFFF)
    return x0 ^ x1


def _sample_kernel(logits_ref, temps_ref, out_ref, bv_ref, bi_ref):
    j = pl.program_id(0)

    @pl.when(j == 0)
    def _init():
        bv_ref[...] = jnp.full((B, C), -jnp.inf, jnp.float32)
        bi_ref[...] = jnp.zeros((B, C), jnp.uint32)

    t = temps_ref[...]
    iota_c = jax.lax.broadcasted_iota(jnp.uint32, (B, C), 1)
    row_v = (jax.lax.broadcasted_iota(jnp.uint32, (B, 1), 0)
             * jnp.uint32(V))
    flat_end = row_v + jnp.uint32(V)
    flat0 = (iota_c + (j * BLK).astype(jnp.uint32)) + row_v

    def chunk(k, carry, masked):
        bv, bi, flat = carry

        bits = _threefry_bits(flat)
        fb = (bits >> jnp.uint32(9)) | jnp.uint32(0x3F800000)
        floats = jax.lax.bitcast_convert_type(fb, jnp.float32) - jnp.float32(1.0)
        u = floats + _TINY
        lg = logits_ref[:, pl.ds(k * C, C)]
        val = lg / t - jnp.log(-jnp.log(u))
        if masked:
            val = jnp.where(flat < flat_end, val, jnp.float32(-jnp.inf))

        upd = val > bv
        return (jnp.where(upd, val, bv), jnp.where(upd, flat, bi),
                flat + jnp.uint32(C))

    carry0 = (bv_ref[...], bi_ref[...], flat0)
    bv, bi, _ = jax.lax.cond(
        j == NBLK - 1,
        lambda c: jax.lax.fori_loop(
            0, NCHUNK, lambda k, cc: chunk(k, cc, True), c, unroll=32),
        lambda c: jax.lax.fori_loop(
            0, NCHUNK, lambda k, cc: chunk(k, cc, False), c, unroll=32),
        carry0)
    bv_ref[...] = bv
    bi_ref[...] = bi

    @pl.when(j == NBLK - 1)
    def _fin():
        m = jnp.max(bv, axis=1, keepdims=True)
        cand = jnp.where(bv == m, bi.astype(jnp.int32), jnp.int32(0x7FFFFFFF))
        out_ref[...] = (jnp.min(cand, axis=1, keepdims=True)
                        - row_v.astype(jnp.int32))


def kernel(logits, temperatures):
    out = pl.pallas_call(
        _sample_kernel,
        grid=(NBLK,),
        in_specs=[
            pl.BlockSpec((B, BLK), lambda j: (0, j)),
            pl.BlockSpec((B, 1), lambda j: (0, 0)),
        ],
        out_specs=pl.BlockSpec((B, 1), lambda j: (0, 0)),
        out_shape=jax.ShapeDtypeStruct((B, 1), jnp.int32),
        scratch_shapes=[
            pltpu.VMEM((B, C), jnp.float32),
            pltpu.VMEM((B, C), jnp.uint32),
        ],
        compiler_params=pltpu.CompilerParams(
            dimension_semantics=("arbitrary",),
        ),
    )(logits, temperatures.reshape(B, 1))
    return out.reshape(B)
